# baseline (speedup 1.0000x reference)
import functools

import jax
import jax.numpy as jnp
from jax import lax
from jax.experimental import pallas as pl
from jax.experimental.pallas import tpu as pltpu
from jax.experimental.pallas import tpu_sc as plsc

NG = 50000
NM = 10000
NMP = 10240
ND = 2048
E = 500000
D = 128
Q = 8192

NC = 2
NS = 16
NW = NC * NS
K = 128

CA = 128
EPAD = NW * CA * K
SEG = 8
NSEG = CA // SEG
BB = 8
RSEG = 18432
RROW = RSEG // K

AROWS = NMP
BROWS = NMP
CNTB = 8448

_mesh = plsc.VectorSubcoreMesh(core_axis_name="c", subcore_axis_name="s")


def _zero_rows(rows_v):
    z = jnp.zeros((16,), jnp.float32)

    def body(r, _):
        for j in range(D // 16):
            rows_v[r, pl.ds(j * 16, 16)] = z
        return 0

    lax.fori_loop(0, K, body, 0)


def _zero_1d(ref, n):
    z = jnp.zeros((16,), jnp.float32)

    def body(i, _):
        ref[pl.ds(i * 16, 16)] = z
        return 0

    lax.fori_loop(0, n // 16, body, 0)


def _sc_gm(src_hbm, dst_hbm, hg_hbm, sums_hbm, cnts_hbm,
           srcb_v, dstb_v, rows0_v, rows1_v, cnt_loc, acc_sh, sem):
    c = lax.axis_index("c")
    s = lax.axis_index("s")
    wid = s * NC + c
    rows = (rows0_v, rows1_v)

    _zero_rows(rows0_v)
    for k in range(AROWS // (16 * 128)):
        pltpu.sync_copy(rows0_v, acc_sh.at[pl.ds(s * (AROWS // 16) + k * 128, 128)])
    _zero_1d(cnt_loc, AROWS)
    plsc.subcore_barrier()

    ones = jnp.ones((16,), jnp.float32)

    def batch(b, _):
        rowbase = wid * (CA // BB) * BB + b * BB
        pltpu.sync_copy(src_hbm.at[pl.ds(rowbase, BB)], srcb_v)
        pltpu.sync_copy(dst_hbm.at[pl.ds(rowbase, BB)], dstb_v)
        for ch in range(BB):
            for j in range(K // 16):
                dv = dstb_v[ch, pl.ds(j * 16, 16)]
                plsc.addupdate_scatter(cnt_loc, [dv], ones)
        h = pltpu.async_copy(hg_hbm.at[srcb_v.at[0]], rows[0], sem)
        for ch in range(BB):
            h.wait()
            if ch + 1 < BB:
                h = pltpu.async_copy(hg_hbm.at[srcb_v.at[ch + 1]],
                                     rows[(ch + 1) % 2], sem)
            pltpu.sync_copy(rows[ch % 2], acc_sh.at[dstb_v.at[ch]], add=True)
        return 0

    lax.fori_loop(0, CA // BB, batch, 0)
    plsc.subcore_barrier()

    for k in range(AROWS // (16 * 128)):
        r = s * (AROWS // 16) + k * 128
        pltpu.sync_copy(acc_sh.at[pl.ds(r, 128)], rows0_v)
        pltpu.sync_copy(rows0_v, sums_hbm.at[pl.ds(c * AROWS + r, 128)])
    pltpu.sync_copy(cnt_loc, cnts_hbm.at[wid])


def _run_sc_gm(src2, dst2, hg):
    f = functools.partial(
        pl.kernel, mesh=_mesh,
        compiler_params=pltpu.CompilerParams(needs_layout_passes=False),
        out_type=[jax.ShapeDtypeStruct((2 * AROWS, D), jnp.float32),
                  jax.ShapeDtypeStruct((NW, AROWS), jnp.float32)],
        scratch_types=[pltpu.VMEM((BB, K), jnp.int32),
                       pltpu.VMEM((BB, K), jnp.int32),
                       pltpu.VMEM((K, D), jnp.float32),
                       pltpu.VMEM((K, D), jnp.float32),
                       pltpu.VMEM((AROWS,), jnp.float32),
                       pltpu.VMEM_SHARED((AROWS, D), jnp.float32),
                       pltpu.SemaphoreType.DMA],
    )(_sc_gm)
    return f(src2, dst2, hg)


def _sc_slot(dst_hbm, src_hbm, gidx_hbm, cslot_hbm, csrc_hbm, rep_hbm,
             nc_hbm, dst_v, src_v, slot_v, cs_slot, cs_src, table_v, gq_v,
             sem):
    c = lax.axis_index("c")
    s = lax.axis_index("s")
    wid = s * NC + c

    neg = jnp.full((16,), -1, jnp.int32)

    def mset(i, _):
        table_v[pl.ds(i * 16, 16)] = neg
        return 0

    lax.fori_loop(0, (NG + 16) // 16, mset, 0)
    pltpu.sync_copy(gidx_hbm, gq_v)
    lanes = lax.iota(jnp.int32, 16)

    def scat(i, _):
        iv = gq_v[pl.ds(i * 16, 16)]
        plsc.store_scatter(table_v, [iv], lanes + i * 16)
        return 0

    lax.fori_loop(0, Q // 16, scat, 0)

    dum_sl = jnp.full((16,), Q, jnp.int32)
    dum_sr = jnp.zeros((16,), jnp.int32)
    rbase = wid * RSEG

    def segment(g, off, _=None):
        def chunk(ch, cs_off):
            base = (wid * CA + g * SEG + ch) * K
            pltpu.sync_copy(dst_hbm.at[pl.ds(base, K)], dst_v)
            pltpu.sync_copy(src_hbm.at[pl.ds(base, K)], src_v)
            for j in range(K // 16):
                dv = dst_v[pl.ds(j * 16, 16)]
                sl = plsc.load_gather(table_v, [dv])
                msk = sl >= 0
                co = pl.multiple_of(cs_off, 8)
                cs_slot[pl.ds(co, 16)] = dum_sl
                cs_src[pl.ds(co, 16)] = dum_sr
                plsc.store_compressed(cs_slot.at[pl.ds(co, 16)], sl,
                                      mask=msk)
                plsc.store_compressed(cs_src.at[pl.ds(co, 16)],
                                      src_v[pl.ds(j * 16, 16)], mask=msk)
                n = plsc.all_reduce_population_count(msk)[0]
                cs_off = cs_off + jnp.bitwise_and(n + 7, -8)
            return cs_off

        cs_off = lax.fori_loop(0, SEG, chunk, 0)
        fo = pl.multiple_of(rbase + off, 8)
        pltpu.sync_copy(cs_slot.at[pl.ds(0, SEG * K)],
                        cslot_hbm.at[pl.ds(fo, SEG * K)])
        pltpu.sync_copy(cs_src.at[pl.ds(0, SEG * K)],
                        csrc_hbm.at[pl.ds(fo, SEG * K)])
        return off + cs_off

    off = lax.fori_loop(0, NSEG, segment, 0)

    for m in range(576 // 16):
        cs_slot[pl.ds(m * 16, 16)] = dum_sl
        cs_src[pl.ds(m * 16, 16)] = dum_sr
    for t in range(2):
        fo = pl.multiple_of(rbase + off + t * 576, 8)
        pltpu.sync_copy(cs_slot.at[pl.ds(0, 576)],
                        cslot_hbm.at[pl.ds(fo, 576)])
        pltpu.sync_copy(cs_src.at[pl.ds(0, 576)],
                        csrc_hbm.at[pl.ds(fo, 576)])

    offv = jnp.full((16,), 1, jnp.int32) * off
    for m in range(K // 16):
        slot_v[pl.ds(m * 16, 16)] = offv
    pltpu.sync_copy(slot_v, nc_hbm.at[pl.ds(wid * K, K)])

    @pl.when(c == 0)
    def _():
        def rchunk(k, _):
            qb = s * (Q // NS) + k * K

            def rsub(j, _):
                iv = gq_v[pl.ds(qb + j * 16, 16)]
                slot_v[pl.ds(j * 16, 16)] = plsc.load_gather(table_v, [iv])
                return 0

            lax.fori_loop(0, K // 16, rsub, 0)
            pltpu.sync_copy(slot_v, rep_hbm.at[pl.ds(qb, K)])
            return 0

        lax.fori_loop(0, (Q // NS) // K, rchunk, 0)


def _run_sc_slot(dst, src, gidx):
    f = functools.partial(
        pl.kernel, mesh=_mesh,
        compiler_params=pltpu.CompilerParams(needs_layout_passes=False),
        out_type=[jax.ShapeDtypeStruct((NW * RSEG,), jnp.int32),
                  jax.ShapeDtypeStruct((NW * RSEG,), jnp.int32),
                  jax.ShapeDtypeStruct((Q,), jnp.int32),
                  jax.ShapeDtypeStruct((NW * K,), jnp.int32)],
        scratch_types=[pltpu.VMEM((K,), jnp.int32),
                       pltpu.VMEM((K,), jnp.int32),
                       pltpu.VMEM((K,), jnp.int32),
                       pltpu.VMEM((SEG * K + 32,), jnp.int32),
                       pltpu.VMEM((SEG * K + 32,), jnp.int32),
                       pltpu.VMEM((NG + 16,), jnp.int32),
                       pltpu.VMEM((Q,), jnp.int32),
                       pltpu.SemaphoreType.DMA],
    )(_sc_slot)
    return f(dst, src, gidx)


def _sc_mg(src_hbm, slote_hbm, nc_hbm, t_hbm, sums_hbm, cnts_hbm,
           srcb_v, slotb_v, rows0_v, rows1_v, ncv, cnt_loc, acc_sh, sem):
    c = lax.axis_index("c")
    s = lax.axis_index("s")
    wid = s * NC + c
    rows = (rows0_v, rows1_v)

    _zero_rows(rows0_v)
    for k in range(BROWS // (16 * 128)):
        pltpu.sync_copy(rows0_v, acc_sh.at[pl.ds(s * (BROWS // 16) + k * 128, 128)])
    _zero_1d(cnt_loc, CNTB)
    plsc.subcore_barrier()

    ones = jnp.ones((16,), jnp.float32)
    coff = c * NMP
    lanes = lax.iota(jnp.int32, 16)

    for r in range(2):
        reg = 2 * s + r
        pltpu.sync_copy(nc_hbm.at[pl.ds(reg * K, K)], ncv)
        n_el = ncv[pl.ds(0, 16)][0]

        def batch(b, _):
            @pl.when(b * (BB * K) < n_el)
            def _():
                rowbase = reg * RROW + b * BB
                pltpu.sync_copy(src_hbm.at[pl.ds(rowbase, BB)], srcb_v)
                pltpu.sync_copy(slote_hbm.at[pl.ds(rowbase, BB)], slotb_v)
                for ch in range(BB):
                    for j in range(K // 16):
                        sl = slotb_v[ch, pl.ds(j * 16, 16)]
                        sl = jnp.where(sl == Q, sl + (j * 16) + lanes, sl)
                        slotb_v[ch, pl.ds(j * 16, 16)] = sl
                        plsc.addupdate_scatter(cnt_loc, [sl], ones)
                        srcb_v[ch, pl.ds(j * 16, 16)] = (
                            srcb_v[ch, pl.ds(j * 16, 16)] + coff)
                h = pltpu.async_copy(t_hbm.at[srcb_v.at[0]], rows[0], sem)
                for ch in range(BB):
                    h.wait()
                    if ch + 1 < BB:
                        h = pltpu.async_copy(t_hbm.at[srcb_v.at[ch + 1]],
                                             rows[(ch + 1) % 2], sem)
                    pltpu.sync_copy(rows[ch % 2], acc_sh.at[slotb_v.at[ch]],
                                    add=True)

            return 0

        lax.fori_loop(0, RROW // BB, batch, 0)

    plsc.subcore_barrier()

    for k in range((Q // NS) // K):
        r = s * (Q // NS) + k * K
        pltpu.sync_copy(acc_sh.at[pl.ds(r, K)], rows0_v)
        pltpu.sync_copy(rows0_v, sums_hbm.at[pl.ds(c * Q + r, K)])
    pltpu.sync_copy(cnt_loc, cnts_hbm.at[wid])


def _run_sc_mg(src, slote, nc, t):
    f = functools.partial(
        pl.kernel, mesh=_mesh,
        compiler_params=pltpu.CompilerParams(needs_layout_passes=False),
        out_type=[jax.ShapeDtypeStruct((2 * Q, D), jnp.float32),
                  jax.ShapeDtypeStruct((NW, CNTB), jnp.float32)],
        scratch_types=[pltpu.VMEM((BB, K), jnp.int32),
                       pltpu.VMEM((BB, K), jnp.int32),
                       pltpu.VMEM((K, D), jnp.float32),
                       pltpu.VMEM((K, D), jnp.float32),
                       pltpu.VMEM((K,), jnp.int32),
                       pltpu.VMEM((CNTB,), jnp.float32),
                       pltpu.VMEM_SHARED((BROWS, D), jnp.float32),
                       pltpu.SemaphoreType.DMA],
    )(_sc_mg)
    return f(src.reshape(-1, K), slote.reshape(-1, K), nc, t)


def _sc_q(rep_hbm, cnt_hbm, sums_hbm, gidx_hbm, didx_hbm, hg_hbm, hd_hbm,
          a0_hbm, a1_hbm, hgq_hbm, hdq_hbm,
          rv, rv2, gv, inv_v, rows_v, cnt_all, sem):
    c = lax.axis_index("c")
    s = lax.axis_index("s")
    wid = s * NC + c

    pltpu.sync_copy(cnt_hbm, cnt_all)

    def norm_store(out_hbm, qb):
        def row(r, _):
            sc = inv_v[pl.ds(r, 16)][0]
            for j in range(D // 16):
                rows_v[r, pl.ds(j * 16, 16)] = rows_v[r, pl.ds(j * 16, 16)] * sc
            return 0

        lax.fori_loop(0, K, row, 0)
        pltpu.sync_copy(rows_v, out_hbm.at[pl.ds(qb, K)])

    def chunk(i, _):
        qb = wid * (Q // NW) + i * K
        pltpu.sync_copy(rep_hbm.at[pl.ds(qb, K)], rv)
        for j in range(K // 16):
            rj = rv[pl.ds(j * 16, 16)]
            cj = plsc.load_gather(cnt_all, [rj])
            inv_v[pl.ds(j * 16, 16)] = 1.0 / jnp.maximum(cj, 1.0)
            rv2[pl.ds(j * 16, 16)] = rj + Q
        pltpu.async_copy(sums_hbm.at[rv], rows_v, sem).wait()
        norm_store(a0_hbm, qb)
        pltpu.async_copy(sums_hbm.at[rv2], rows_v, sem).wait()
        norm_store(a1_hbm, qb)
        pltpu.sync_copy(gidx_hbm.at[pl.ds(qb, K)], gv)
        pltpu.async_copy(hg_hbm.at[gv], rows_v, sem).wait()
        pltpu.sync_copy(rows_v, hgq_hbm.at[pl.ds(qb, K)])
        pltpu.sync_copy(didx_hbm.at[pl.ds(qb, K)], gv)
        pltpu.async_copy(hd_hbm.at[gv], rows_v, sem).wait()
        pltpu.sync_copy(rows_v, hdq_hbm.at[pl.ds(qb, K)])
        return 0

    lax.fori_loop(0, (Q // NW) // K, chunk, 0)


def _run_sc_q(rep, cnt, sums, gidx, didx, hg, hd):
    f = functools.partial(
        pl.kernel, mesh=_mesh,
        compiler_params=pltpu.CompilerParams(needs_layout_passes=False),
        out_type=[jax.ShapeDtypeStruct((Q, D), jnp.float32),
                  jax.ShapeDtypeStruct((Q, D), jnp.float32),
                  jax.ShapeDtypeStruct((Q, D), jnp.float32),
                  jax.ShapeDtypeStruct((Q, D), jnp.float32)],
        scratch_types=[pltpu.VMEM((K,), jnp.int32),
                       pltpu.VMEM((K,), jnp.int32),
                       pltpu.VMEM((K,), jnp.int32),
                       pltpu.VMEM((K + 16,), jnp.float32),
                       pltpu.VMEM((K, D), jnp.float32),
                       pltpu.VMEM((CNTB,), jnp.float32),
                       pltpu.SemaphoreType.DMA],
    )(_sc_q)
    return f(rep, cnt, sums, gidx, didx, hg, hd)


def _gelu(x):
    return 0.5 * x * (1.0 + lax.erf(x * 0.7071067811865476))


def _l2n(x):
    n = jnp.sqrt(jnp.sum(x * x, axis=-1, keepdims=True))
    return x / jnp.maximum(n, 1e-12)


def _lnorm(x, w, b):
    mu = jnp.mean(x, axis=-1, keepdims=True)
    var = jnp.mean((x - mu) ** 2, axis=-1, keepdims=True)
    return (x - mu) * lax.rsqrt(var + 1e-5) * w + b


def _proj_body(act, x_ref, w_ref, b_ref, o_ref):
    y = jnp.dot(x_ref[...], w_ref[...], preferred_element_type=jnp.float32)
    y = y + b_ref[...]
    o_ref[...] = _gelu(y) if act else y


def _run_proj(x, w, b, act, blk):
    n = x.shape[0]
    return pl.pallas_call(
        functools.partial(_proj_body, act),
        grid=(n // blk,),
        in_specs=[pl.BlockSpec((blk, D), lambda i: (i, 0)),
                  pl.BlockSpec((D, D), lambda i: (0, 0)),
                  pl.BlockSpec((1, D), lambda i: (0, 0))],
        out_specs=pl.BlockSpec((blk, D), lambda i: (i, 0)),
        out_shape=jax.ShapeDtypeStruct((n, D), jnp.float32),
    )(x, w, b.reshape(1, D))


def _mech_body(p0_ref, p1_ref, cnt_ref, hm_ref, wl_ref, bl_ref, wr_ref,
               lw_ref, lb_ref, t_ref):
    cnt = jnp.sum(cnt_ref[...], axis=0)[:, None]
    agg = (p0_ref[0] + p1_ref[0]) / jnp.maximum(cnt, 1.0)
    hm = hm_ref[...]
    out_m = _l2n(jnp.dot(agg, wl_ref[...], preferred_element_type=jnp.float32)
                 + bl_ref[...]
                 + jnp.dot(hm, wr_ref[...], preferred_element_type=jnp.float32))
    t_ref[0] = hm
    t_ref[1] = _lnorm(_gelu(out_m), lw_ref[...], lb_ref[...])


def _run_mech(sums, cnts, hm, wl, bl, wr, lw, lb):
    blk = 1024
    s3 = sums.reshape(2, AROWS, D)
    return pl.pallas_call(
        _mech_body,
        grid=(NMP // blk,),
        in_specs=[pl.BlockSpec((1, blk, D), lambda i: (0, i, 0)),
                  pl.BlockSpec((1, blk, D), lambda i: (1, i, 0)),
                  pl.BlockSpec((NW, blk), lambda i: (0, i)),
                  pl.BlockSpec((blk, D), lambda i: (i, 0)),
                  pl.BlockSpec((D, D), lambda i: (0, 0)),
                  pl.BlockSpec((1, D), lambda i: (0, 0)),
                  pl.BlockSpec((D, D), lambda i: (0, 0)),
                  pl.BlockSpec((1, D), lambda i: (0, 0)),
                  pl.BlockSpec((1, D), lambda i: (0, 0))],
        out_specs=pl.BlockSpec((2, blk, D), lambda i: (0, i, 0)),
        out_shape=jax.ShapeDtypeStruct((2, NMP, D), jnp.float32),
    )(s3, s3, cnts, hm, wl, bl.reshape(1, D), wr, lw.reshape(1, D),
      lb.reshape(1, D))


def _cntred_body(c_ref, o_ref):
    o_ref[...] = 0.5 * jnp.sum(c_ref[...], axis=0, keepdims=True)


def _run_cntred(cnts):
    return pl.pallas_call(
        _cntred_body,
        grid=(1,),
        in_specs=[pl.BlockSpec((NW, CNTB), lambda i: (0, 0))],
        out_specs=pl.BlockSpec((1, CNTB), lambda i: (0, 0)),
        out_shape=jax.ShapeDtypeStruct((1, CNTB), jnp.float32),
    )(cnts)


def _final_body(a0_ref, a1_ref, hgq_ref, hdq_ref, wl1_ref, bl1_ref, wr1_ref,
                wl2_ref, bl2_ref, wr2_ref, lw_ref, lb_ref, wb_ref, o_ref):
    out_g = _l2n(jnp.dot(a0_ref[...], wl1_ref[...],
                         preferred_element_type=jnp.float32)
                 + bl1_ref[...]
                 + jnp.dot(hgq_ref[...], wr1_ref[...],
                           preferred_element_type=jnp.float32))
    h_g = _lnorm(_gelu(out_g), lw_ref[...], lb_ref[...])
    ge = _l2n(jnp.dot(a1_ref[...], wl2_ref[...],
                      preferred_element_type=jnp.float32)
              + bl2_ref[...]
              + jnp.dot(h_g, wr2_ref[...], preferred_element_type=jnp.float32))
    o_ref[...] = jnp.sum(ge * wb_ref[...] * hdq_ref[...], axis=-1)[None, :]


def _run_final(a0, a1, hgq, hdq, wl1, bl1, wr1, wl2, bl2, wr2, lw, lb, wb):
    blk = 1024
    full = lambda i: (0, 0)
    return pl.pallas_call(
        _final_body,
        grid=(Q // blk,),
        in_specs=[pl.BlockSpec((blk, D), lambda i: (i, 0)),
                  pl.BlockSpec((blk, D), lambda i: (i, 0)),
                  pl.BlockSpec((blk, D), lambda i: (i, 0)),
                  pl.BlockSpec((blk, D), lambda i: (i, 0)),
                  pl.BlockSpec((D, D), full),
                  pl.BlockSpec((1, D), full),
                  pl.BlockSpec((D, D), full),
                  pl.BlockSpec((D, D), full),
                  pl.BlockSpec((1, D), full),
                  pl.BlockSpec((D, D), full),
                  pl.BlockSpec((1, D), full),
                  pl.BlockSpec((1, D), full),
                  pl.BlockSpec((1, D), full)],
        out_specs=pl.BlockSpec((1, blk), lambda i: (0, i)),
        out_shape=jax.ShapeDtypeStruct((1, Q), jnp.float32),
    )(a0, a1, hgq, hdq, wl1, bl1.reshape(1, D), wr1, wl2, bl2.reshape(1, D),
      wr2, lw.reshape(1, D), lb.reshape(1, D), wb.reshape(1, D))


def kernel(gene_x, mech_x, drug_x, src_gm, dst_gm, src_mg, dst_mg, gene_idx,
           drug_idx, Wg, bg, Wm, bm, Wd, bd, Wl1a, bl1a, Wr1a, Wl1b, bl1b,
           Wr1b, Wl2b, bl2b, Wr2b, lng_w, lng_b, lnm_w, lnm_b, Wbil):
    i32 = jnp.int32
    pe = EPAD - E
    src_gm_p = jnp.concatenate([src_gm.astype(i32), jnp.zeros((pe,), i32)])
    dst_gm_p = jnp.concatenate([dst_gm.astype(i32), jnp.full((pe,), NM, i32)])
    src_mg_p = jnp.concatenate([src_mg.astype(i32), jnp.zeros((pe,), i32)])
    dst_mg_p = jnp.concatenate([dst_mg.astype(i32), jnp.full((pe,), NG, i32)])
    gidx = gene_idx.astype(i32)
    didx = drug_idx.astype(i32)

    h_gene = _run_proj(gene_x, Wg, bg, True, 1000)
    mech_pad = jnp.pad(mech_x, ((0, NMP - NM), (0, 0)))
    h_mech = _run_proj(mech_pad, Wm, bm, True, 1024)
    h_drug = _run_proj(drug_x, Wd, bd, False, 1024)

    sums_a, cnts_a = _run_sc_gm(src_gm_p.reshape(-1, K),
                                dst_gm_p.reshape(-1, K), h_gene)
    t = _run_mech(sums_a, cnts_a, h_mech, Wl1a, bl1a, Wr1a, lnm_w, lnm_b)
    t_flat = t.reshape(2 * NMP, D)

    cslot, csrc, rep, nc = _run_sc_slot(dst_mg_p, src_mg_p, gidx)
    sums_b, cnts_b = _run_sc_mg(csrc, cslot, nc, t_flat)
    cnt_tot = _run_cntred(cnts_b).reshape(CNTB)

    a0, a1, hgq, hdq = _run_sc_q(rep, cnt_tot, sums_b, gidx, didx,
                                 h_gene, h_drug)
    score = _run_final(a0, a1, hgq, hdq, Wl1b, bl1b, Wr1b, Wl2b, bl2b, Wr2b,
                       lng_w, lng_b, Wbil)
    return score.reshape(Q)


# speedup vs baseline: 4.8441x; 4.8441x over previous
import functools

import jax
import jax.numpy as jnp
from jax import lax
from jax.experimental import pallas as pl
from jax.experimental.pallas import tpu as pltpu
from jax.experimental.pallas import tpu_sc as plsc

NG = 50000
NM = 10000
NMP = 10240
ND = 2048
E = 500000
D = 128
Q = 8192

NC = 2
NS = 16
NW = NC * NS
K = 128

CA = 128
EPAD = NW * CA * K
SEG = 8
NSEG = CA // SEG
BB = 8
RSEG = 18432
RROW = RSEG // K

AROWS = NMP
BROWS = NMP
CNTB = 8448

_mesh = plsc.VectorSubcoreMesh(core_axis_name="c", subcore_axis_name="s")


def _zero_rows(rows_v):
    z = jnp.zeros((16,), jnp.float32)

    def body(r, _):
        for j in range(D // 16):
            rows_v[r, pl.ds(j * 16, 16)] = z
        return 0

    lax.fori_loop(0, K, body, 0)


def _zero_1d(ref, n):
    z = jnp.zeros((16,), jnp.float32)

    def body(i, _):
        ref[pl.ds(i * 16, 16)] = z
        return 0

    lax.fori_loop(0, n // 16, body, 0)


def _sc_gm(src_hbm, dst_hbm, hg_hbm, sums_hbm, cnts_hbm,
           srcb_v, dstb_v, rows0_v, rows1_v, cnt_loc, acc_sh, sem):
    c = lax.axis_index("c")
    s = lax.axis_index("s")
    wid = s * NC + c
    rows = (rows0_v, rows1_v)

    _zero_rows(rows0_v)
    for k in range(AROWS // (16 * 128)):
        pltpu.sync_copy(rows0_v, acc_sh.at[pl.ds(s * (AROWS // 16) + k * 128, 128)])
    _zero_1d(cnt_loc, AROWS)
    plsc.subcore_barrier()

    ones = jnp.ones((16,), jnp.float32)

    def batch(b, _):
        rowbase = wid * (CA // BB) * BB + b * BB
        pltpu.sync_copy(src_hbm.at[pl.ds(rowbase, BB)], srcb_v)
        pltpu.sync_copy(dst_hbm.at[pl.ds(rowbase, BB)], dstb_v)
        for ch in range(BB):
            for j in range(K // 16):
                dv = dstb_v[ch, pl.ds(j * 16, 16)]
                plsc.addupdate_scatter(cnt_loc, [dv], ones)
        h = pltpu.async_copy(hg_hbm.at[srcb_v.at[0]], rows[0], sem)
        for ch in range(BB):
            h.wait()
            if ch + 1 < BB:
                h = pltpu.async_copy(hg_hbm.at[srcb_v.at[ch + 1]],
                                     rows[(ch + 1) % 2], sem)
            pltpu.sync_copy(rows[ch % 2], acc_sh.at[dstb_v.at[ch]], add=True)
        return 0

    lax.fori_loop(0, CA // BB, batch, 0)
    plsc.subcore_barrier()

    for k in range(AROWS // (16 * 128)):
        r = s * (AROWS // 16) + k * 128
        pltpu.sync_copy(acc_sh.at[pl.ds(r, 128)], rows0_v)
        pltpu.sync_copy(rows0_v, sums_hbm.at[pl.ds(c * AROWS + r, 128)])
    pltpu.sync_copy(cnt_loc, cnts_hbm.at[wid])


def _run_sc_gm(src2, dst2, hg):
    f = functools.partial(
        pl.kernel, mesh=_mesh,
        compiler_params=pltpu.CompilerParams(needs_layout_passes=False),
        out_type=[jax.ShapeDtypeStruct((2 * AROWS, D), jnp.float32),
                  jax.ShapeDtypeStruct((NW, AROWS), jnp.float32)],
        scratch_types=[pltpu.VMEM((BB, K), jnp.int32),
                       pltpu.VMEM((BB, K), jnp.int32),
                       pltpu.VMEM((K, D), jnp.float32),
                       pltpu.VMEM((K, D), jnp.float32),
                       pltpu.VMEM((AROWS,), jnp.float32),
                       pltpu.VMEM_SHARED((AROWS, D), jnp.float32),
                       pltpu.SemaphoreType.DMA],
    )(_sc_gm)
    return f(src2, dst2, hg)


def _sc_slot(dst_hbm, src_hbm, gidx_hbm, cslot_hbm, csrc_hbm, rep_hbm,
             nc_hbm, dst_v, src_v, slot_v, cs_slot, cs_src, table_v, gq_v,
             sem):
    c = lax.axis_index("c")
    s = lax.axis_index("s")
    wid = s * NC + c

    neg = jnp.full((16,), -1, jnp.int32)

    def mset(i, _):
        table_v[pl.ds(i * 16, 16)] = neg
        return 0

    lax.fori_loop(0, (NG + 16) // 16, mset, 0)
    pltpu.sync_copy(gidx_hbm, gq_v)
    lanes = lax.iota(jnp.int32, 16)

    def scat(i, _):
        iv = gq_v[pl.ds(i * 16, 16)]
        plsc.store_scatter(table_v, [iv], lanes + i * 16)
        return 0

    lax.fori_loop(0, Q // 16, scat, 0)

    dum_sl = jnp.full((16,), Q, jnp.int32)
    dum_sr = jnp.zeros((16,), jnp.int32)
    rbase = wid * RSEG

    def segment(g, off, _=None):
        def chunk(ch, cs_off):
            base = (wid * CA + g * SEG + ch) * K
            pltpu.sync_copy(dst_hbm.at[pl.ds(base, K)], dst_v)
            pltpu.sync_copy(src_hbm.at[pl.ds(base, K)], src_v)
            for j in range(K // 16):
                dv = dst_v[pl.ds(j * 16, 16)]
                sl = plsc.load_gather(table_v, [dv])
                msk = sl >= 0
                co = pl.multiple_of(cs_off, 8)
                cs_slot[pl.ds(co, 16)] = dum_sl
                cs_src[pl.ds(co, 16)] = dum_sr
                plsc.store_compressed(cs_slot.at[pl.ds(co, 16)], sl,
                                      mask=msk)
                plsc.store_compressed(cs_src.at[pl.ds(co, 16)],
                                      src_v[pl.ds(j * 16, 16)], mask=msk)
                n = plsc.all_reduce_population_count(msk)[0]
                cs_off = cs_off + jnp.bitwise_and(n + 7, -8)
            return cs_off

        cs_off = lax.fori_loop(0, SEG, chunk, 0)
        fo = pl.multiple_of(rbase + off, 8)
        pltpu.sync_copy(cs_slot.at[pl.ds(0, SEG * K)],
                        cslot_hbm.at[pl.ds(fo, SEG * K)])
        pltpu.sync_copy(cs_src.at[pl.ds(0, SEG * K)],
                        csrc_hbm.at[pl.ds(fo, SEG * K)])
        return off + cs_off

    off = lax.fori_loop(0, NSEG, segment, 0)

    for m in range(576 // 16):
        cs_slot[pl.ds(m * 16, 16)] = dum_sl
        cs_src[pl.ds(m * 16, 16)] = dum_sr
    for t in range(2):
        fo = pl.multiple_of(rbase + off + t * 576, 8)
        pltpu.sync_copy(cs_slot.at[pl.ds(0, 576)],
                        cslot_hbm.at[pl.ds(fo, 576)])
        pltpu.sync_copy(cs_src.at[pl.ds(0, 576)],
                        csrc_hbm.at[pl.ds(fo, 576)])

    offv = jnp.full((16,), 1, jnp.int32) * off
    for m in range(K // 16):
        slot_v[pl.ds(m * 16, 16)] = offv
    pltpu.sync_copy(slot_v, nc_hbm.at[pl.ds(wid * K, K)])

    @pl.when(c == 0)
    def _():
        def rchunk(k, _):
            qb = s * (Q // NS) + k * K

            def rsub(j, _):
                iv = gq_v[pl.ds(qb + j * 16, 16)]
                slot_v[pl.ds(j * 16, 16)] = plsc.load_gather(table_v, [iv])
                return 0

            lax.fori_loop(0, K // 16, rsub, 0)
            pltpu.sync_copy(slot_v, rep_hbm.at[pl.ds(qb, K)])
            return 0

        lax.fori_loop(0, (Q // NS) // K, rchunk, 0)


def _run_sc_slot(dst, src, gidx):
    f = functools.partial(
        pl.kernel, mesh=_mesh,
        compiler_params=pltpu.CompilerParams(needs_layout_passes=False),
        out_type=[jax.ShapeDtypeStruct((NW * RSEG,), jnp.int32),
                  jax.ShapeDtypeStruct((NW * RSEG,), jnp.int32),
                  jax.ShapeDtypeStruct((Q,), jnp.int32),
                  jax.ShapeDtypeStruct((NW * K,), jnp.int32)],
        scratch_types=[pltpu.VMEM((K,), jnp.int32),
                       pltpu.VMEM((K,), jnp.int32),
                       pltpu.VMEM((K,), jnp.int32),
                       pltpu.VMEM((SEG * K + 32,), jnp.int32),
                       pltpu.VMEM((SEG * K + 32,), jnp.int32),
                       pltpu.VMEM((NG + 16,), jnp.int32),
                       pltpu.VMEM((Q,), jnp.int32),
                       pltpu.SemaphoreType.DMA],
    )(_sc_slot)
    return f(dst, src, gidx)


def _sc_mg(src_hbm, slote_hbm, nc_hbm, t_hbm, sums_hbm, cnts_hbm,
           srcb_v, slotb_v, rows0_v, rows1_v, ncv, cnt_loc, acc_sh, sem):
    c = lax.axis_index("c")
    s = lax.axis_index("s")
    wid = s * NC + c
    rows = (rows0_v, rows1_v)

    _zero_rows(rows0_v)
    for k in range(BROWS // (16 * 128)):
        pltpu.sync_copy(rows0_v, acc_sh.at[pl.ds(s * (BROWS // 16) + k * 128, 128)])
    _zero_1d(cnt_loc, CNTB)
    plsc.subcore_barrier()

    ones = jnp.ones((16,), jnp.float32)
    coff = c * NMP
    lanes = lax.iota(jnp.int32, 16)

    for r in range(2):
        reg = 2 * s + r
        pltpu.sync_copy(nc_hbm.at[pl.ds(reg * K, K)], ncv)
        n_el = ncv[pl.ds(0, 16)][0]

        def batch(b, _):
            @pl.when(b * (BB * K) < n_el)
            def _():
                rowbase = reg * RROW + b * BB
                pltpu.sync_copy(src_hbm.at[pl.ds(rowbase, BB)], srcb_v)
                pltpu.sync_copy(slote_hbm.at[pl.ds(rowbase, BB)], slotb_v)
                for ch in range(BB):
                    for j in range(K // 16):
                        sl = slotb_v[ch, pl.ds(j * 16, 16)]
                        msk = sl == Q
                        spread = (j * 16) + lanes
                        sl = jnp.where(msk, sl + spread, sl)
                        slotb_v[ch, pl.ds(j * 16, 16)] = sl
                        plsc.addupdate_scatter(cnt_loc, [sl], ones)
                        sv = srcb_v[ch, pl.ds(j * 16, 16)]
                        srcb_v[ch, pl.ds(j * 16, 16)] = (
                            jnp.where(msk, NM + spread, sv) + coff)
                h = pltpu.async_copy(t_hbm.at[srcb_v.at[0]], rows[0], sem)
                for ch in range(BB):
                    h.wait()
                    if ch + 1 < BB:
                        h = pltpu.async_copy(t_hbm.at[srcb_v.at[ch + 1]],
                                             rows[(ch + 1) % 2], sem)
                    pltpu.sync_copy(rows[ch % 2], acc_sh.at[slotb_v.at[ch]],
                                    add=True)

            return 0

        lax.fori_loop(0, RROW // BB, batch, 0)

    plsc.subcore_barrier()

    for k in range((Q // NS) // K):
        r = s * (Q // NS) + k * K
        pltpu.sync_copy(acc_sh.at[pl.ds(r, K)], rows0_v)
        pltpu.sync_copy(rows0_v, sums_hbm.at[pl.ds(c * Q + r, K)])
    pltpu.sync_copy(cnt_loc, cnts_hbm.at[wid])


def _run_sc_mg(src, slote, nc, t):
    f = functools.partial(
        pl.kernel, mesh=_mesh,
        compiler_params=pltpu.CompilerParams(needs_layout_passes=False),
        out_type=[jax.ShapeDtypeStruct((2 * Q, D), jnp.float32),
                  jax.ShapeDtypeStruct((NW, CNTB), jnp.float32)],
        scratch_types=[pltpu.VMEM((BB, K), jnp.int32),
                       pltpu.VMEM((BB, K), jnp.int32),
                       pltpu.VMEM((K, D), jnp.float32),
                       pltpu.VMEM((K, D), jnp.float32),
                       pltpu.VMEM((K,), jnp.int32),
                       pltpu.VMEM((CNTB,), jnp.float32),
                       pltpu.VMEM_SHARED((BROWS, D), jnp.float32),
                       pltpu.SemaphoreType.DMA],
    )(_sc_mg)
    return f(src.reshape(-1, K), slote.reshape(-1, K), nc, t)


def _sc_q(rep_hbm, cnt_hbm, sums_hbm, gidx_hbm, didx_hbm, hg_hbm, hd_hbm,
          a0_hbm, a1_hbm, hgq_hbm, hdq_hbm,
          rv, rv2, gv, inv_v, rows_v, cnt_all, sem):
    c = lax.axis_index("c")
    s = lax.axis_index("s")
    wid = s * NC + c

    pltpu.sync_copy(cnt_hbm, cnt_all)

    def norm_store(out_hbm, qb):
        def row(r, _):
            sc = inv_v[pl.ds(r, 16)][0]
            for j in range(D // 16):
                rows_v[r, pl.ds(j * 16, 16)] = rows_v[r, pl.ds(j * 16, 16)] * sc
            return 0

        lax.fori_loop(0, K, row, 0)
        pltpu.sync_copy(rows_v, out_hbm.at[pl.ds(qb, K)])

    def chunk(i, _):
        qb = wid * (Q // NW) + i * K
        pltpu.sync_copy(rep_hbm.at[pl.ds(qb, K)], rv)
        for j in range(K // 16):
            rj = rv[pl.ds(j * 16, 16)]
            cj = plsc.load_gather(cnt_all, [rj])
            inv_v[pl.ds(j * 16, 16)] = 1.0 / jnp.maximum(cj, 1.0)
            rv2[pl.ds(j * 16, 16)] = rj + Q
        pltpu.async_copy(sums_hbm.at[rv], rows_v, sem).wait()
        norm_store(a0_hbm, qb)
        pltpu.async_copy(sums_hbm.at[rv2], rows_v, sem).wait()
        norm_store(a1_hbm, qb)
        pltpu.sync_copy(gidx_hbm.at[pl.ds(qb, K)], gv)
        pltpu.async_copy(hg_hbm.at[gv], rows_v, sem).wait()
        pltpu.sync_copy(rows_v, hgq_hbm.at[pl.ds(qb, K)])
        pltpu.sync_copy(didx_hbm.at[pl.ds(qb, K)], gv)
        pltpu.async_copy(hd_hbm.at[gv], rows_v, sem).wait()
        pltpu.sync_copy(rows_v, hdq_hbm.at[pl.ds(qb, K)])
        return 0

    lax.fori_loop(0, (Q // NW) // K, chunk, 0)


def _run_sc_q(rep, cnt, sums, gidx, didx, hg, hd):
    f = functools.partial(
        pl.kernel, mesh=_mesh,
        compiler_params=pltpu.CompilerParams(needs_layout_passes=False),
        out_type=[jax.ShapeDtypeStruct((Q, D), jnp.float32),
                  jax.ShapeDtypeStruct((Q, D), jnp.float32),
                  jax.ShapeDtypeStruct((Q, D), jnp.float32),
                  jax.ShapeDtypeStruct((Q, D), jnp.float32)],
        scratch_types=[pltpu.VMEM((K,), jnp.int32),
                       pltpu.VMEM((K,), jnp.int32),
                       pltpu.VMEM((K,), jnp.int32),
                       pltpu.VMEM((K + 16,), jnp.float32),
                       pltpu.VMEM((K, D), jnp.float32),
                       pltpu.VMEM((CNTB,), jnp.float32),
                       pltpu.SemaphoreType.DMA],
    )(_sc_q)
    return f(rep, cnt, sums, gidx, didx, hg, hd)


def _gelu(x):
    return 0.5 * x * (1.0 + lax.erf(x * 0.7071067811865476))


def _l2n(x):
    n = jnp.sqrt(jnp.sum(x * x, axis=-1, keepdims=True))
    return x / jnp.maximum(n, 1e-12)


def _lnorm(x, w, b):
    mu = jnp.mean(x, axis=-1, keepdims=True)
    var = jnp.mean((x - mu) ** 2, axis=-1, keepdims=True)
    return (x - mu) * lax.rsqrt(var + 1e-5) * w + b


def _proj_body(act, x_ref, w_ref, b_ref, o_ref):
    y = jnp.dot(x_ref[...], w_ref[...], preferred_element_type=jnp.float32)
    y = y + b_ref[...]
    o_ref[...] = _gelu(y) if act else y


def _run_proj(x, w, b, act, blk):
    n = x.shape[0]
    return pl.pallas_call(
        functools.partial(_proj_body, act),
        grid=(n // blk,),
        in_specs=[pl.BlockSpec((blk, D), lambda i: (i, 0)),
                  pl.BlockSpec((D, D), lambda i: (0, 0)),
                  pl.BlockSpec((1, D), lambda i: (0, 0))],
        out_specs=pl.BlockSpec((blk, D), lambda i: (i, 0)),
        out_shape=jax.ShapeDtypeStruct((n, D), jnp.float32),
    )(x, w, b.reshape(1, D))


def _mech_body(p0_ref, p1_ref, cnt_ref, hm_ref, wl_ref, bl_ref, wr_ref,
               lw_ref, lb_ref, t_ref):
    cnt = jnp.sum(cnt_ref[...], axis=0)[:, None]
    agg = (p0_ref[0] + p1_ref[0]) / jnp.maximum(cnt, 1.0)
    hm = hm_ref[...]
    out_m = _l2n(jnp.dot(agg, wl_ref[...], preferred_element_type=jnp.float32)
                 + bl_ref[...]
                 + jnp.dot(hm, wr_ref[...], preferred_element_type=jnp.float32))
    t_ref[0] = hm
    t_ref[1] = _lnorm(_gelu(out_m), lw_ref[...], lb_ref[...])


def _run_mech(sums, cnts, hm, wl, bl, wr, lw, lb):
    blk = 1024
    s3 = sums.reshape(2, AROWS, D)
    return pl.pallas_call(
        _mech_body,
        grid=(NMP // blk,),
        in_specs=[pl.BlockSpec((1, blk, D), lambda i: (0, i, 0)),
                  pl.BlockSpec((1, blk, D), lambda i: (1, i, 0)),
                  pl.BlockSpec((NW, blk), lambda i: (0, i)),
                  pl.BlockSpec((blk, D), lambda i: (i, 0)),
                  pl.BlockSpec((D, D), lambda i: (0, 0)),
                  pl.BlockSpec((1, D), lambda i: (0, 0)),
                  pl.BlockSpec((D, D), lambda i: (0, 0)),
                  pl.BlockSpec((1, D), lambda i: (0, 0)),
                  pl.BlockSpec((1, D), lambda i: (0, 0))],
        out_specs=pl.BlockSpec((2, blk, D), lambda i: (0, i, 0)),
        out_shape=jax.ShapeDtypeStruct((2, NMP, D), jnp.float32),
    )(s3, s3, cnts, hm, wl, bl.reshape(1, D), wr, lw.reshape(1, D),
      lb.reshape(1, D))


def _cntred_body(c_ref, o_ref):
    o_ref[...] = 0.5 * jnp.sum(c_ref[...], axis=0, keepdims=True)


def _run_cntred(cnts):
    return pl.pallas_call(
        _cntred_body,
        grid=(1,),
        in_specs=[pl.BlockSpec((NW, CNTB), lambda i: (0, 0))],
        out_specs=pl.BlockSpec((1, CNTB), lambda i: (0, 0)),
        out_shape=jax.ShapeDtypeStruct((1, CNTB), jnp.float32),
    )(cnts)


def _final_body(a0_ref, a1_ref, hgq_ref, hdq_ref, wl1_ref, bl1_ref, wr1_ref,
                wl2_ref, bl2_ref, wr2_ref, lw_ref, lb_ref, wb_ref, o_ref):
    out_g = _l2n(jnp.dot(a0_ref[...], wl1_ref[...],
                         preferred_element_type=jnp.float32)
                 + bl1_ref[...]
                 + jnp.dot(hgq_ref[...], wr1_ref[...],
                           preferred_element_type=jnp.float32))
    h_g = _lnorm(_gelu(out_g), lw_ref[...], lb_ref[...])
    ge = _l2n(jnp.dot(a1_ref[...], wl2_ref[...],
                      preferred_element_type=jnp.float32)
              + bl2_ref[...]
              + jnp.dot(h_g, wr2_ref[...], preferred_element_type=jnp.float32))
    o_ref[...] = jnp.sum(ge * wb_ref[...] * hdq_ref[...], axis=-1)[None, :]


def _run_final(a0, a1, hgq, hdq, wl1, bl1, wr1, wl2, bl2, wr2, lw, lb, wb):
    blk = 1024
    full = lambda i: (0, 0)
    return pl.pallas_call(
        _final_body,
        grid=(Q // blk,),
        in_specs=[pl.BlockSpec((blk, D), lambda i: (i, 0)),
                  pl.BlockSpec((blk, D), lambda i: (i, 0)),
                  pl.BlockSpec((blk, D), lambda i: (i, 0)),
                  pl.BlockSpec((blk, D), lambda i: (i, 0)),
                  pl.BlockSpec((D, D), full),
                  pl.BlockSpec((1, D), full),
                  pl.BlockSpec((D, D), full),
                  pl.BlockSpec((D, D), full),
                  pl.BlockSpec((1, D), full),
                  pl.BlockSpec((D, D), full),
                  pl.BlockSpec((1, D), full),
                  pl.BlockSpec((1, D), full),
                  pl.BlockSpec((1, D), full)],
        out_specs=pl.BlockSpec((1, blk), lambda i: (0, i)),
        out_shape=jax.ShapeDtypeStruct((1, Q), jnp.float32),
    )(a0, a1, hgq, hdq, wl1, bl1.reshape(1, D), wr1, wl2, bl2.reshape(1, D),
      wr2, lw.reshape(1, D), lb.reshape(1, D), wb.reshape(1, D))


def kernel(gene_x, mech_x, drug_x, src_gm, dst_gm, src_mg, dst_mg, gene_idx,
           drug_idx, Wg, bg, Wm, bm, Wd, bd, Wl1a, bl1a, Wr1a, Wl1b, bl1b,
           Wr1b, Wl2b, bl2b, Wr2b, lng_w, lng_b, lnm_w, lnm_b, Wbil):
    i32 = jnp.int32
    pe = EPAD - E
    src_gm_p = jnp.concatenate([src_gm.astype(i32), jnp.zeros((pe,), i32)])
    dst_gm_p = jnp.concatenate([dst_gm.astype(i32), jnp.full((pe,), NM, i32)])
    src_mg_p = jnp.concatenate([src_mg.astype(i32), jnp.zeros((pe,), i32)])
    dst_mg_p = jnp.concatenate([dst_mg.astype(i32), jnp.full((pe,), NG, i32)])
    gidx = gene_idx.astype(i32)
    didx = drug_idx.astype(i32)

    h_gene = _run_proj(gene_x, Wg, bg, True, 1000)
    mech_pad = jnp.pad(mech_x, ((0, NMP - NM), (0, 0)))
    h_mech = _run_proj(mech_pad, Wm, bm, True, 1024)
    h_drug = _run_proj(drug_x, Wd, bd, False, 1024)

    sums_a, cnts_a = _run_sc_gm(src_gm_p.reshape(-1, K),
                                dst_gm_p.reshape(-1, K), h_gene)
    t = _run_mech(sums_a, cnts_a, h_mech, Wl1a, bl1a, Wr1a, lnm_w, lnm_b)
    t_flat = t.reshape(2 * NMP, D)

    cslot, csrc, rep, nc = _run_sc_slot(dst_mg_p, src_mg_p, gidx)
    sums_b, cnts_b = _run_sc_mg(csrc, cslot, nc, t_flat)
    cnt_tot = _run_cntred(cnts_b).reshape(CNTB)

    a0, a1, hgq, hdq = _run_sc_q(rep, cnt_tot, sums_b, gidx, didx,
                                 h_gene, h_drug)
    score = _run_final(a0, a1, hgq, hdq, Wl1b, bl1b, Wr1b, Wl2b, bl2b, Wr2b,
                       lng_w, lng_b, Wbil)
    return score.reshape(Q)


# speedup vs baseline: 10.0014x; 2.0647x over previous
import functools

import jax
import jax.numpy as jnp
from jax import lax
from jax.experimental import pallas as pl
from jax.experimental.pallas import tpu as pltpu
from jax.experimental.pallas import tpu_sc as plsc

NG = 50000
NM = 10000
NMP = 10240
ND = 2048
E = 500000
D = 128
Q = 8192

NC = 2
NS = 16
NW = NC * NS
K = 128

CA = 128
EPAD = NW * CA * K
SEG = 8
NSEG = CA // SEG
BB = 8
RSEG = 18432
RROW = RSEG // K

AROWS = NMP
BROWS = NMP
CNTB = 8448

_mesh = plsc.VectorSubcoreMesh(core_axis_name="c", subcore_axis_name="s")


def _zero_rows(rows_v):
    z = jnp.zeros((16,), jnp.float32)

    def body(r, _):
        for j in range(D // 16):
            rows_v[r, pl.ds(j * 16, 16)] = z
        return 0

    lax.fori_loop(0, K, body, 0)


def _zero_1d(ref, n):
    z = jnp.zeros((16,), jnp.float32)

    def body(i, _):
        ref[pl.ds(i * 16, 16)] = z
        return 0

    lax.fori_loop(0, n // 16, body, 0)


def _sc_gm(src_hbm, dst_hbm, hg_hbm, sums_hbm, cnts_hbm,
           srcb_v, dstb_v, rows0_v, rows1_v, cnt_loc, acc_sh, sem):
    c = lax.axis_index("c")
    s = lax.axis_index("s")
    wid = s * NC + c
    rows = (rows0_v, rows1_v)

    _zero_rows(rows0_v)
    for k in range(AROWS // (16 * 128)):
        pltpu.sync_copy(rows0_v, acc_sh.at[pl.ds(s * (AROWS // 16) + k * 128, 128)])
    _zero_1d(cnt_loc, AROWS)
    plsc.subcore_barrier()

    ones = jnp.ones((16,), jnp.float32)
    lanes = lax.iota(jnp.int32, 16)

    def batch(b, _):
        rowbase = wid * (CA // BB) * BB + b * BB
        pltpu.sync_copy(src_hbm.at[pl.ds(rowbase, BB)], srcb_v)
        pltpu.sync_copy(dst_hbm.at[pl.ds(rowbase, BB)], dstb_v)
        for ch in range(BB):
            for j in range(K // 16):
                dv = dstb_v[ch, pl.ds(j * 16, 16)]
                msk = dv == NM
                spread = (j * 16) + lanes
                dv = jnp.where(msk, dv + spread, dv)
                dstb_v[ch, pl.ds(j * 16, 16)] = dv
                sv = srcb_v[ch, pl.ds(j * 16, 16)]
                srcb_v[ch, pl.ds(j * 16, 16)] = jnp.where(msk, spread, sv)
                plsc.addupdate_scatter(cnt_loc, [dv], ones)
        h = pltpu.async_copy(hg_hbm.at[srcb_v.at[0]], rows[0], sem)
        for ch in range(BB):
            h.wait()
            if ch + 1 < BB:
                h = pltpu.async_copy(hg_hbm.at[srcb_v.at[ch + 1]],
                                     rows[(ch + 1) % 2], sem)
            pltpu.sync_copy(rows[ch % 2], acc_sh.at[dstb_v.at[ch]], add=True)
        return 0

    lax.fori_loop(0, CA // BB, batch, 0)
    plsc.subcore_barrier()

    for k in range(AROWS // (16 * 128)):
        r = s * (AROWS // 16) + k * 128
        pltpu.sync_copy(acc_sh.at[pl.ds(r, 128)], rows0_v)
        pltpu.sync_copy(rows0_v, sums_hbm.at[pl.ds(c * AROWS + r, 128)])
    pltpu.sync_copy(cnt_loc, cnts_hbm.at[wid])


def _run_sc_gm(src2, dst2, hg):
    f = functools.partial(
        pl.kernel, mesh=_mesh,
        compiler_params=pltpu.CompilerParams(needs_layout_passes=False),
        out_type=[jax.ShapeDtypeStruct((2 * AROWS, D), jnp.float32),
                  jax.ShapeDtypeStruct((NW, AROWS), jnp.float32)],
        scratch_types=[pltpu.VMEM((BB, K), jnp.int32),
                       pltpu.VMEM((BB, K), jnp.int32),
                       pltpu.VMEM((K, D), jnp.float32),
                       pltpu.VMEM((K, D), jnp.float32),
                       pltpu.VMEM((AROWS,), jnp.float32),
                       pltpu.VMEM_SHARED((AROWS, D), jnp.float32),
                       pltpu.SemaphoreType.DMA],
    )(_sc_gm)
    return f(src2, dst2, hg)


def _sc_slot(dst_hbm, src_hbm, gidx_hbm, cslot_hbm, csrc_hbm, rep_hbm,
             nc_hbm, dst_v, src_v, slot_v, cs_slot, cs_src, table_v, gq_v,
             sem):
    c = lax.axis_index("c")
    s = lax.axis_index("s")
    wid = s * NC + c

    neg = jnp.full((16,), -1, jnp.int32)

    def mset(i, _):
        table_v[pl.ds(i * 16, 16)] = neg
        return 0

    lax.fori_loop(0, (NG + 16) // 16, mset, 0)
    pltpu.sync_copy(gidx_hbm, gq_v)
    lanes = lax.iota(jnp.int32, 16)

    def scat(i, _):
        iv = gq_v[pl.ds(i * 16, 16)]
        plsc.store_scatter(table_v, [iv], lanes + i * 16)
        return 0

    lax.fori_loop(0, Q // 16, scat, 0)

    dum_sl = jnp.full((16,), Q, jnp.int32)
    dum_sr = jnp.zeros((16,), jnp.int32)
    rbase = wid * RSEG

    def segment(g, off, _=None):
        def chunk(ch, cs_off):
            base = (wid * CA + g * SEG + ch) * K
            pltpu.sync_copy(dst_hbm.at[pl.ds(base, K)], dst_v)
            pltpu.sync_copy(src_hbm.at[pl.ds(base, K)], src_v)
            for j in range(K // 16):
                dv = dst_v[pl.ds(j * 16, 16)]
                sl = plsc.load_gather(table_v, [dv])
                msk = sl >= 0
                co = pl.multiple_of(cs_off, 8)
                cs_slot[pl.ds(co, 16)] = dum_sl
                cs_src[pl.ds(co, 16)] = dum_sr
                plsc.store_compressed(cs_slot.at[pl.ds(co, 16)], sl,
                                      mask=msk)
                plsc.store_compressed(cs_src.at[pl.ds(co, 16)],
                                      src_v[pl.ds(j * 16, 16)], mask=msk)
                n = plsc.all_reduce_population_count(msk)[0]
                cs_off = cs_off + jnp.bitwise_and(n + 7, -8)
            return cs_off

        cs_off = lax.fori_loop(0, SEG, chunk, 0)
        fo = pl.multiple_of(rbase + off, 8)
        pltpu.sync_copy(cs_slot.at[pl.ds(0, SEG * K)],
                        cslot_hbm.at[pl.ds(fo, SEG * K)])
        pltpu.sync_copy(cs_src.at[pl.ds(0, SEG * K)],
                        csrc_hbm.at[pl.ds(fo, SEG * K)])
        return off + cs_off

    off = lax.fori_loop(0, NSEG, segment, 0)

    for m in range(576 // 16):
        cs_slot[pl.ds(m * 16, 16)] = dum_sl
        cs_src[pl.ds(m * 16, 16)] = dum_sr
    for t in range(2):
        fo = pl.multiple_of(rbase + off + t * 576, 8)
        pltpu.sync_copy(cs_slot.at[pl.ds(0, 576)],
                        cslot_hbm.at[pl.ds(fo, 576)])
        pltpu.sync_copy(cs_src.at[pl.ds(0, 576)],
                        csrc_hbm.at[pl.ds(fo, 576)])

    offv = jnp.full((16,), 1, jnp.int32) * off
    for m in range(K // 16):
        slot_v[pl.ds(m * 16, 16)] = offv
    pltpu.sync_copy(slot_v, nc_hbm.at[pl.ds(wid * K, K)])

    @pl.when(c == 0)
    def _():
        def rchunk(k, _):
            qb = s * (Q // NS) + k * K

            def rsub(j, _):
                iv = gq_v[pl.ds(qb + j * 16, 16)]
                slot_v[pl.ds(j * 16, 16)] = plsc.load_gather(table_v, [iv])
                return 0

            lax.fori_loop(0, K // 16, rsub, 0)
            pltpu.sync_copy(slot_v, rep_hbm.at[pl.ds(qb, K)])
            return 0

        lax.fori_loop(0, (Q // NS) // K, rchunk, 0)


def _run_sc_slot(dst, src, gidx):
    f = functools.partial(
        pl.kernel, mesh=_mesh,
        compiler_params=pltpu.CompilerParams(needs_layout_passes=False),
        out_type=[jax.ShapeDtypeStruct((NW * RSEG,), jnp.int32),
                  jax.ShapeDtypeStruct((NW * RSEG,), jnp.int32),
                  jax.ShapeDtypeStruct((Q,), jnp.int32),
                  jax.ShapeDtypeStruct((NW * K,), jnp.int32)],
        scratch_types=[pltpu.VMEM((K,), jnp.int32),
                       pltpu.VMEM((K,), jnp.int32),
                       pltpu.VMEM((K,), jnp.int32),
                       pltpu.VMEM((SEG * K + 32,), jnp.int32),
                       pltpu.VMEM((SEG * K + 32,), jnp.int32),
                       pltpu.VMEM((NG + 16,), jnp.int32),
                       pltpu.VMEM((Q,), jnp.int32),
                       pltpu.SemaphoreType.DMA],
    )(_sc_slot)
    return f(dst, src, gidx)


def _sc_mg(src_hbm, slote_hbm, nc_hbm, t_hbm, sums_hbm, cnts_hbm,
           srcb_v, slotb_v, rows0_v, rows1_v, ncv, cnt_loc, acc_sh, sem):
    c = lax.axis_index("c")
    s = lax.axis_index("s")
    wid = s * NC + c
    rows = (rows0_v, rows1_v)

    _zero_rows(rows0_v)
    for k in range(BROWS // (16 * 128)):
        pltpu.sync_copy(rows0_v, acc_sh.at[pl.ds(s * (BROWS // 16) + k * 128, 128)])
    _zero_1d(cnt_loc, CNTB)
    plsc.subcore_barrier()

    ones = jnp.ones((16,), jnp.float32)
    coff = c * NMP
    lanes = lax.iota(jnp.int32, 16)

    for r in range(2):
        reg = 2 * s + r
        pltpu.sync_copy(nc_hbm.at[pl.ds(reg * K, K)], ncv)
        n_el = ncv[pl.ds(0, 16)][0]

        def batch(b, _):
            @pl.when(b * (BB * K) < n_el)
            def _():
                rowbase = reg * RROW + b * BB
                pltpu.sync_copy(src_hbm.at[pl.ds(rowbase, BB)], srcb_v)
                pltpu.sync_copy(slote_hbm.at[pl.ds(rowbase, BB)], slotb_v)
                for ch in range(BB):
                    for j in range(K // 16):
                        sl = slotb_v[ch, pl.ds(j * 16, 16)]
                        msk = sl == Q
                        spread = (j * 16) + lanes
                        sl = jnp.where(msk, sl + spread, sl)
                        slotb_v[ch, pl.ds(j * 16, 16)] = sl
                        plsc.addupdate_scatter(cnt_loc, [sl], ones)
                        sv = srcb_v[ch, pl.ds(j * 16, 16)]
                        srcb_v[ch, pl.ds(j * 16, 16)] = (
                            jnp.where(msk, NM + spread, sv) + coff)
                h = pltpu.async_copy(t_hbm.at[srcb_v.at[0]], rows[0], sem)
                for ch in range(BB):
                    h.wait()
                    if ch + 1 < BB:
                        h = pltpu.async_copy(t_hbm.at[srcb_v.at[ch + 1]],
                                             rows[(ch + 1) % 2], sem)
                    pltpu.sync_copy(rows[ch % 2], acc_sh.at[slotb_v.at[ch]],
                                    add=True)

            return 0

        lax.fori_loop(0, RROW // BB, batch, 0)

    plsc.subcore_barrier()

    for k in range((Q // NS) // K):
        r = s * (Q // NS) + k * K
        pltpu.sync_copy(acc_sh.at[pl.ds(r, K)], rows0_v)
        pltpu.sync_copy(rows0_v, sums_hbm.at[pl.ds(c * Q + r, K)])
    pltpu.sync_copy(cnt_loc, cnts_hbm.at[wid])


def _run_sc_mg(src, slote, nc, t):
    f = functools.partial(
        pl.kernel, mesh=_mesh,
        compiler_params=pltpu.CompilerParams(needs_layout_passes=False),
        out_type=[jax.ShapeDtypeStruct((2 * Q, D), jnp.float32),
                  jax.ShapeDtypeStruct((NW, CNTB), jnp.float32)],
        scratch_types=[pltpu.VMEM((BB, K), jnp.int32),
                       pltpu.VMEM((BB, K), jnp.int32),
                       pltpu.VMEM((K, D), jnp.float32),
                       pltpu.VMEM((K, D), jnp.float32),
                       pltpu.VMEM((K,), jnp.int32),
                       pltpu.VMEM((CNTB,), jnp.float32),
                       pltpu.VMEM_SHARED((BROWS, D), jnp.float32),
                       pltpu.SemaphoreType.DMA],
    )(_sc_mg)
    return f(src.reshape(-1, K), slote.reshape(-1, K), nc, t)


def _sc_q(rep_hbm, cnt_hbm, sums_hbm, gidx_hbm, didx_hbm, hg_hbm, hd_hbm,
          a0_hbm, a1_hbm, hgq_hbm, hdq_hbm,
          rv, rv2, gv, inv_v, rows_v, cnt_all, sem):
    c = lax.axis_index("c")
    s = lax.axis_index("s")
    wid = s * NC + c

    pltpu.sync_copy(cnt_hbm, cnt_all)

    def norm_store(out_hbm, qb):
        def row(r, _):
            sc = inv_v[pl.ds(r, 16)][0]
            for j in range(D // 16):
                rows_v[r, pl.ds(j * 16, 16)] = rows_v[r, pl.ds(j * 16, 16)] * sc
            return 0

        lax.fori_loop(0, K, row, 0)
        pltpu.sync_copy(rows_v, out_hbm.at[pl.ds(qb, K)])

    def chunk(i, _):
        qb = wid * (Q // NW) + i * K
        pltpu.sync_copy(rep_hbm.at[pl.ds(qb, K)], rv)
        for j in range(K // 16):
            rj = rv[pl.ds(j * 16, 16)]
            cj = plsc.load_gather(cnt_all, [rj])
            inv_v[pl.ds(j * 16, 16)] = 1.0 / jnp.maximum(cj, 1.0)
            rv2[pl.ds(j * 16, 16)] = rj + Q
        pltpu.async_copy(sums_hbm.at[rv], rows_v, sem).wait()
        norm_store(a0_hbm, qb)
        pltpu.async_copy(sums_hbm.at[rv2], rows_v, sem).wait()
        norm_store(a1_hbm, qb)
        pltpu.sync_copy(gidx_hbm.at[pl.ds(qb, K)], gv)
        pltpu.async_copy(hg_hbm.at[gv], rows_v, sem).wait()
        pltpu.sync_copy(rows_v, hgq_hbm.at[pl.ds(qb, K)])
        pltpu.sync_copy(didx_hbm.at[pl.ds(qb, K)], gv)
        pltpu.async_copy(hd_hbm.at[gv], rows_v, sem).wait()
        pltpu.sync_copy(rows_v, hdq_hbm.at[pl.ds(qb, K)])
        return 0

    lax.fori_loop(0, (Q // NW) // K, chunk, 0)


def _run_sc_q(rep, cnt, sums, gidx, didx, hg, hd):
    f = functools.partial(
        pl.kernel, mesh=_mesh,
        compiler_params=pltpu.CompilerParams(needs_layout_passes=False),
        out_type=[jax.ShapeDtypeStruct((Q, D), jnp.float32),
                  jax.ShapeDtypeStruct((Q, D), jnp.float32),
                  jax.ShapeDtypeStruct((Q, D), jnp.float32),
                  jax.ShapeDtypeStruct((Q, D), jnp.float32)],
        scratch_types=[pltpu.VMEM((K,), jnp.int32),
                       pltpu.VMEM((K,), jnp.int32),
                       pltpu.VMEM((K,), jnp.int32),
                       pltpu.VMEM((K + 16,), jnp.float32),
                       pltpu.VMEM((K, D), jnp.float32),
                       pltpu.VMEM((CNTB,), jnp.float32),
                       pltpu.SemaphoreType.DMA],
    )(_sc_q)
    return f(rep, cnt, sums, gidx, didx, hg, hd)


def _gelu(x):
    return 0.5 * x * (1.0 + lax.erf(x * 0.7071067811865476))


def _l2n(x):
    n = jnp.sqrt(jnp.sum(x * x, axis=-1, keepdims=True))
    return x / jnp.maximum(n, 1e-12)


def _lnorm(x, w, b):
    mu = jnp.mean(x, axis=-1, keepdims=True)
    var = jnp.mean((x - mu) ** 2, axis=-1, keepdims=True)
    return (x - mu) * lax.rsqrt(var + 1e-5) * w + b


def _proj_body(act, x_ref, w_ref, b_ref, o_ref):
    y = jnp.dot(x_ref[...], w_ref[...], preferred_element_type=jnp.float32)
    y = y + b_ref[...]
    o_ref[...] = _gelu(y) if act else y


def _run_proj(x, w, b, act, blk):
    n = x.shape[0]
    return pl.pallas_call(
        functools.partial(_proj_body, act),
        grid=(n // blk,),
        in_specs=[pl.BlockSpec((blk, D), lambda i: (i, 0)),
                  pl.BlockSpec((D, D), lambda i: (0, 0)),
                  pl.BlockSpec((1, D), lambda i: (0, 0))],
        out_specs=pl.BlockSpec((blk, D), lambda i: (i, 0)),
        out_shape=jax.ShapeDtypeStruct((n, D), jnp.float32),
    )(x, w, b.reshape(1, D))


def _mech_body(p0_ref, p1_ref, cnt_ref, hm_ref, wl_ref, bl_ref, wr_ref,
               lw_ref, lb_ref, t_ref):
    cnt = jnp.sum(cnt_ref[...], axis=0)[:, None]
    agg = (p0_ref[0] + p1_ref[0]) / jnp.maximum(cnt, 1.0)
    hm = hm_ref[...]
    out_m = _l2n(jnp.dot(agg, wl_ref[...], preferred_element_type=jnp.float32)
                 + bl_ref[...]
                 + jnp.dot(hm, wr_ref[...], preferred_element_type=jnp.float32))
    t_ref[0] = hm
    t_ref[1] = _lnorm(_gelu(out_m), lw_ref[...], lb_ref[...])


def _run_mech(sums, cnts, hm, wl, bl, wr, lw, lb):
    blk = 1024
    s3 = sums.reshape(2, AROWS, D)
    return pl.pallas_call(
        _mech_body,
        grid=(NMP // blk,),
        in_specs=[pl.BlockSpec((1, blk, D), lambda i: (0, i, 0)),
                  pl.BlockSpec((1, blk, D), lambda i: (1, i, 0)),
                  pl.BlockSpec((NW, blk), lambda i: (0, i)),
                  pl.BlockSpec((blk, D), lambda i: (i, 0)),
                  pl.BlockSpec((D, D), lambda i: (0, 0)),
                  pl.BlockSpec((1, D), lambda i: (0, 0)),
                  pl.BlockSpec((D, D), lambda i: (0, 0)),
                  pl.BlockSpec((1, D), lambda i: (0, 0)),
                  pl.BlockSpec((1, D), lambda i: (0, 0))],
        out_specs=pl.BlockSpec((2, blk, D), lambda i: (0, i, 0)),
        out_shape=jax.ShapeDtypeStruct((2, NMP, D), jnp.float32),
    )(s3, s3, cnts, hm, wl, bl.reshape(1, D), wr, lw.reshape(1, D),
      lb.reshape(1, D))


def _cntred_body(c_ref, o_ref):
    o_ref[...] = 0.5 * jnp.sum(c_ref[...], axis=0, keepdims=True)


def _run_cntred(cnts):
    return pl.pallas_call(
        _cntred_body,
        grid=(1,),
        in_specs=[pl.BlockSpec((NW, CNTB), lambda i: (0, 0))],
        out_specs=pl.BlockSpec((1, CNTB), lambda i: (0, 0)),
        out_shape=jax.ShapeDtypeStruct((1, CNTB), jnp.float32),
    )(cnts)


def _final_body(a0_ref, a1_ref, hgq_ref, hdq_ref, wl1_ref, bl1_ref, wr1_ref,
                wl2_ref, bl2_ref, wr2_ref, lw_ref, lb_ref, wb_ref, o_ref):
    out_g = _l2n(jnp.dot(a0_ref[...], wl1_ref[...],
                         preferred_element_type=jnp.float32)
                 + bl1_ref[...]
                 + jnp.dot(hgq_ref[...], wr1_ref[...],
                           preferred_element_type=jnp.float32))
    h_g = _lnorm(_gelu(out_g), lw_ref[...], lb_ref[...])
    ge = _l2n(jnp.dot(a1_ref[...], wl2_ref[...],
                      preferred_element_type=jnp.float32)
              + bl2_ref[...]
              + jnp.dot(h_g, wr2_ref[...], preferred_element_type=jnp.float32))
    o_ref[...] = jnp.sum(ge * wb_ref[...] * hdq_ref[...], axis=-1)[None, :]


def _run_final(a0, a1, hgq, hdq, wl1, bl1, wr1, wl2, bl2, wr2, lw, lb, wb):
    blk = 1024
    full = lambda i: (0, 0)
    return pl.pallas_call(
        _final_body,
        grid=(Q // blk,),
        in_specs=[pl.BlockSpec((blk, D), lambda i: (i, 0)),
                  pl.BlockSpec((blk, D), lambda i: (i, 0)),
                  pl.BlockSpec((blk, D), lambda i: (i, 0)),
                  pl.BlockSpec((blk, D), lambda i: (i, 0)),
                  pl.BlockSpec((D, D), full),
                  pl.BlockSpec((1, D), full),
                  pl.BlockSpec((D, D), full),
                  pl.BlockSpec((D, D), full),
                  pl.BlockSpec((1, D), full),
                  pl.BlockSpec((D, D), full),
                  pl.BlockSpec((1, D), full),
                  pl.BlockSpec((1, D), full),
                  pl.BlockSpec((1, D), full)],
        out_specs=pl.BlockSpec((1, blk), lambda i: (0, i)),
        out_shape=jax.ShapeDtypeStruct((1, Q), jnp.float32),
    )(a0, a1, hgq, hdq, wl1, bl1.reshape(1, D), wr1, wl2, bl2.reshape(1, D),
      wr2, lw.reshape(1, D), lb.reshape(1, D), wb.reshape(1, D))


def kernel(gene_x, mech_x, drug_x, src_gm, dst_gm, src_mg, dst_mg, gene_idx,
           drug_idx, Wg, bg, Wm, bm, Wd, bd, Wl1a, bl1a, Wr1a, Wl1b, bl1b,
           Wr1b, Wl2b, bl2b, Wr2b, lng_w, lng_b, lnm_w, lnm_b, Wbil):
    i32 = jnp.int32
    pe = EPAD - E
    src_gm_p = jnp.concatenate([src_gm.astype(i32), jnp.zeros((pe,), i32)])
    dst_gm_p = jnp.concatenate([dst_gm.astype(i32), jnp.full((pe,), NM, i32)])
    src_mg_p = jnp.concatenate([src_mg.astype(i32), jnp.zeros((pe,), i32)])
    dst_mg_p = jnp.concatenate([dst_mg.astype(i32), jnp.full((pe,), NG, i32)])
    gidx = gene_idx.astype(i32)
    didx = drug_idx.astype(i32)

    h_gene = _run_proj(gene_x, Wg, bg, True, 1000)
    mech_pad = jnp.pad(mech_x, ((0, NMP - NM), (0, 0)))
    h_mech = _run_proj(mech_pad, Wm, bm, True, 1024)
    h_drug = _run_proj(drug_x, Wd, bd, False, 1024)

    sums_a, cnts_a = _run_sc_gm(src_gm_p.reshape(-1, K),
                                dst_gm_p.reshape(-1, K), h_gene)
    t = _run_mech(sums_a, cnts_a, h_mech, Wl1a, bl1a, Wr1a, lnm_w, lnm_b)
    t_flat = t.reshape(2 * NMP, D)

    cslot, csrc, rep, nc = _run_sc_slot(dst_mg_p, src_mg_p, gidx)
    sums_b, cnts_b = _run_sc_mg(csrc, cslot, nc, t_flat)
    cnt_tot = _run_cntred(cnts_b).reshape(CNTB)

    a0, a1, hgq, hdq = _run_sc_q(rep, cnt_tot, sums_b, gidx, didx,
                                 h_gene, h_drug)
    score = _run_final(a0, a1, hgq, hdq, Wl1b, bl1b, Wr1b, Wl2b, bl2b, Wr2b,
                       lng_w, lng_b, Wbil)
    return score.reshape(Q)
